# baseline (speedup 1.0000x reference)
import functools

import jax
import jax.numpy as jnp
from jax import lax
from jax.experimental import pallas as pl
from jax.experimental.pallas import tpu as pltpu
from jax.experimental.pallas import tpu_sc as plsc

N_USERS = 1000000
N_ITEMS = 100000
D = 64
BATCH = 16384

_info = plsc.get_sparse_core_info()
_NC, _NS, _L = _info.num_cores, _info.num_subcores, _info.num_lanes
_NW = _NC * _NS
_BPW = BATCH // _NW


def _tt_kernel(uid_hbm, iid_hbm, ut_hbm, it_hbm, out_hbm,
               uidx_v, iidx_v, urows_v, irows_v, out_v, usem, isem):
    wid = lax.axis_index("s") * _NC + lax.axis_index("c")
    base = wid * _BPW
    pltpu.sync_copy(uid_hbm.at[pl.ds(base, _BPW)], uidx_v)
    pltpu.sync_copy(iid_hbm.at[pl.ds(base, _BPW)], iidx_v)
    ucp = pltpu.async_copy(ut_hbm.at[uidx_v], urows_v, usem)
    icp = pltpu.async_copy(it_hbm.at[iidx_v], irows_v, isem)
    ucp.wait()
    icp.wait()

    row_iota = lax.iota(jnp.int32, _L)

    def group_body(g, carry):
        rows = g * _L + row_iota

        def d_body(d, acc):
            col = jnp.full((_L,), d, jnp.int32)
            u = plsc.load_gather(urows_v, [rows, col])
            v = plsc.load_gather(irows_v, [rows, col])
            return acc + u * v

        acc = lax.fori_loop(0, D, d_body, jnp.zeros((_L,), jnp.float32),
                            unroll=8)
        out_v[pl.ds(g * _L, _L)] = acc
        return carry

    lax.fori_loop(0, _BPW // _L, group_body, 0)
    pltpu.sync_copy(out_v, out_hbm.at[pl.ds(base, _BPW)])


@jax.jit
def _two_tower(user_ids, item_ids, user_table, item_table):
    mesh = plsc.VectorSubcoreMesh(core_axis_name="c", subcore_axis_name="s")
    f = functools.partial(
        pl.kernel,
        out_type=jax.ShapeDtypeStruct((BATCH,), jnp.float32),
        mesh=mesh,
        scratch_types=[
            pltpu.VMEM((_BPW,), jnp.int32),
            pltpu.VMEM((_BPW,), jnp.int32),
            pltpu.VMEM((_BPW, D), jnp.float32),
            pltpu.VMEM((_BPW, D), jnp.float32),
            pltpu.VMEM((_BPW,), jnp.float32),
            pltpu.SemaphoreType.DMA,
            pltpu.SemaphoreType.DMA,
        ],
        compiler_params=pltpu.CompilerParams(needs_layout_passes=False,
                                             use_tc_tiling_on_sc=False),
    )(_tt_kernel)
    return f(user_ids, item_ids, user_table, item_table)


def kernel(user_ids, item_ids, user_table, item_table):
    return _two_tower(user_ids.astype(jnp.int32), item_ids.astype(jnp.int32),
                      user_table, item_table)


# speedup vs baseline: 1.6050x; 1.6050x over previous
import functools

import jax
import jax.numpy as jnp
from jax import lax
from jax.experimental import pallas as pl
from jax.experimental.pallas import tpu as pltpu
from jax.experimental.pallas import tpu_sc as plsc

N_USERS = 1000000
N_ITEMS = 100000
D = 64
BATCH = 16384

_info = plsc.get_sparse_core_info()
_NC, _NS, _L = _info.num_cores, _info.num_subcores, _info.num_lanes
_NW = _NC * _NS
_BPW = BATCH // _NW
_CH = 128
_NCH = _BPW // _CH


def _tt_kernel(uid_hbm, iid_hbm, ut_hbm, it_hbm, out_hbm,
               uids_v, iids_v, urows0, urows1, irows0, irows1,
               out_v, usem0, usem1, isem0, isem1):
    wid = lax.axis_index("s") * _NC + lax.axis_index("c")
    base = wid * _BPW
    pltpu.sync_copy(uid_hbm.at[pl.ds(base, _BPW)], uids_v)
    pltpu.sync_copy(iid_hbm.at[pl.ds(base, _BPW)], iids_v)

    ubufs, ibufs = (urows0, urows1), (irows0, irows1)
    usems, isems = (usem0, usem1), (isem0, isem1)
    row_iota = lax.iota(jnp.int32, _L)

    def enqueue(c, s):
        ubuf, ibuf, usem, isem = ubufs[s], ibufs[s], usems[s], isems[s]

        def body(g, carry):
            uvec = uids_v[pl.ds(c * _CH + g * _L, _L)]
            ivec = iids_v[pl.ds(c * _CH + g * _L, _L)]
            for j in range(_L):
                pltpu.make_async_copy(
                    ut_hbm.at[pl.ds(uvec[j], 1), :],
                    ubuf.at[pl.ds(g * _L + j, 1), :], usem).start()
                pltpu.make_async_copy(
                    it_hbm.at[pl.ds(ivec[j], 1), :],
                    ibuf.at[pl.ds(g * _L + j, 1), :], isem).start()
            return carry

        lax.fori_loop(0, _CH // _L, body, 0)

    def wait(s):
        pltpu.make_async_copy(ut_hbm.at[pl.ds(0, _CH), :], ubufs[s],
                              usems[s]).wait()
        pltpu.make_async_copy(it_hbm.at[pl.ds(0, _CH), :], ibufs[s],
                              isems[s]).wait()

    def compute(c, s):
        ubuf, ibuf = ubufs[s], ibufs[s]

        def group_body(g, carry):
            rows = g * _L + row_iota

            def d_body(d, acc):
                col = jnp.full((_L,), d, jnp.int32)
                u = plsc.load_gather(ubuf, [rows, col])
                v = plsc.load_gather(ibuf, [rows, col])
                return acc + u * v

            acc = lax.fori_loop(0, D, d_body, jnp.zeros((_L,), jnp.float32),
                                unroll=8)
            out_v[pl.ds(c * _CH + g * _L, _L)] = acc
            return carry

        lax.fori_loop(0, _CH // _L, group_body, 0)

    enqueue(0, 0)
    for c in range(_NCH):
        s = c % 2
        if c + 1 < _NCH:
            enqueue(c + 1, (c + 1) % 2)
        wait(s)
        compute(c, s)

    pltpu.sync_copy(out_v, out_hbm.at[pl.ds(base, _BPW)])


@jax.jit
def _two_tower(user_ids, item_ids, user_table, item_table):
    mesh = plsc.VectorSubcoreMesh(core_axis_name="c", subcore_axis_name="s")
    f = functools.partial(
        pl.kernel,
        out_type=jax.ShapeDtypeStruct((BATCH,), jnp.float32),
        mesh=mesh,
        scratch_types=[
            pltpu.VMEM((_BPW,), jnp.int32),
            pltpu.VMEM((_BPW,), jnp.int32),
            pltpu.VMEM((_CH, D), jnp.float32),
            pltpu.VMEM((_CH, D), jnp.float32),
            pltpu.VMEM((_CH, D), jnp.float32),
            pltpu.VMEM((_CH, D), jnp.float32),
            pltpu.VMEM((_BPW,), jnp.float32),
            pltpu.SemaphoreType.DMA,
            pltpu.SemaphoreType.DMA,
            pltpu.SemaphoreType.DMA,
            pltpu.SemaphoreType.DMA,
        ],
        compiler_params=pltpu.CompilerParams(needs_layout_passes=False),
    )(_tt_kernel)
    return f(user_ids, item_ids, user_table, item_table)


def kernel(user_ids, item_ids, user_table, item_table):
    return _two_tower(user_ids.astype(jnp.int32), item_ids.astype(jnp.int32),
                      user_table, item_table)


# speedup vs baseline: 1.6091x; 1.0026x over previous
import functools

import jax
import jax.numpy as jnp
from jax import lax
from jax.experimental import pallas as pl
from jax.experimental.pallas import tpu as pltpu
from jax.experimental.pallas import tpu_sc as plsc

N_USERS = 1000000
N_ITEMS = 100000
D = 64
BATCH = 16384

_info = plsc.get_sparse_core_info()
_NC, _NS, _L = _info.num_cores, _info.num_subcores, _info.num_lanes
_NW = _NC * _NS
_BPW = BATCH // _NW
_CH = 128
_NCH = _BPW // _CH


def _tt_kernel(uid_hbm, iid_hbm, ut_hbm, it_hbm, out_hbm,
               uids_v, iids_v, urows0, urows1, irows0, irows1,
               out_v, usem0, usem1, isem0, isem1):
    wid = lax.axis_index("s") * _NC + lax.axis_index("c")
    base = wid * _BPW
    pltpu.sync_copy(uid_hbm.at[pl.ds(base, _BPW)], uids_v)
    pltpu.sync_copy(iid_hbm.at[pl.ds(base, _BPW)], iids_v)

    ubufs, ibufs = (urows0, urows1), (irows0, irows1)
    usems, isems = (usem0, usem1), (isem0, isem1)
    row_iota = lax.iota(jnp.int32, _L)

    def enqueue(c, s):
        ubuf, ibuf, usem, isem = ubufs[s], ibufs[s], usems[s], isems[s]

        def body(g, carry):
            uvec = uids_v[pl.ds(c * _CH + g * _L, _L)]
            ivec = iids_v[pl.ds(c * _CH + g * _L, _L)]
            for j in range(_L):
                pltpu.make_async_copy(
                    ut_hbm.at[pl.ds(uvec[j], 1), :],
                    ubuf.at[pl.ds(g * _L + j, 1), :], usem).start()
                pltpu.make_async_copy(
                    it_hbm.at[pl.ds(ivec[j], 1), :],
                    ibuf.at[pl.ds(g * _L + j, 1), :], isem).start()
            return carry

        lax.fori_loop(0, _CH // _L, body, 0)

    def wait(s):
        pltpu.make_async_copy(ut_hbm.at[pl.ds(0, _CH), :], ubufs[s],
                              usems[s]).wait()
        pltpu.make_async_copy(it_hbm.at[pl.ds(0, _CH), :], ibufs[s],
                              isems[s]).wait()

    def compute(c, s):
        ubuf, ibuf = ubufs[s], ibufs[s]

        def group_body(g, carry):
            rows = g * _L + row_iota

            def d_body(d, acc):
                col = jnp.full((_L,), d, jnp.int32)
                u = plsc.load_gather(ubuf, [rows, col])
                v = plsc.load_gather(ibuf, [rows, col])
                return acc + u * v

            acc = lax.fori_loop(0, D, d_body, jnp.zeros((_L,), jnp.float32),
                                unroll=8)
            out_v[pl.ds(c * _CH + g * _L, _L)] = acc
            return carry

        lax.fori_loop(0, _CH // _L, group_body, 0)

    with jax.named_scope("enqueue0"):
        enqueue(0, 0)
    for c in range(_NCH):
        s = c % 2
        if c + 1 < _NCH:
            with jax.named_scope(f"enqueue{c + 1}"):
                enqueue(c + 1, (c + 1) % 2)
        with jax.named_scope(f"wait{c}"):
            wait(s)
        with jax.named_scope(f"compute{c}"):
            compute(c, s)

    pltpu.sync_copy(out_v, out_hbm.at[pl.ds(base, _BPW)])


@jax.jit
def _two_tower(user_ids, item_ids, user_table, item_table):
    mesh = plsc.VectorSubcoreMesh(core_axis_name="c", subcore_axis_name="s")
    f = functools.partial(
        pl.kernel,
        out_type=jax.ShapeDtypeStruct((BATCH,), jnp.float32),
        mesh=mesh,
        scratch_types=[
            pltpu.VMEM((_BPW,), jnp.int32),
            pltpu.VMEM((_BPW,), jnp.int32),
            pltpu.VMEM((_CH, D), jnp.float32),
            pltpu.VMEM((_CH, D), jnp.float32),
            pltpu.VMEM((_CH, D), jnp.float32),
            pltpu.VMEM((_CH, D), jnp.float32),
            pltpu.VMEM((_BPW,), jnp.float32),
            pltpu.SemaphoreType.DMA,
            pltpu.SemaphoreType.DMA,
            pltpu.SemaphoreType.DMA,
            pltpu.SemaphoreType.DMA,
        ],
        compiler_params=pltpu.CompilerParams(needs_layout_passes=False),
    )(_tt_kernel)
    return f(user_ids, item_ids, user_table, item_table)


def kernel(user_ids, item_ids, user_table, item_table):
    return _two_tower(user_ids.astype(jnp.int32), item_ids.astype(jnp.int32),
                      user_table, item_table)


# speedup vs baseline: 1.6100x; 1.0005x over previous
import functools

import jax
import jax.numpy as jnp
from jax import lax
from jax.experimental import pallas as pl
from jax.experimental.pallas import tpu as pltpu
from jax.experimental.pallas import tpu_sc as plsc

N_USERS = 1000000
N_ITEMS = 100000
D = 64
BATCH = 16384

_info = plsc.get_sparse_core_info()
_NC, _NS, _L = _info.num_cores, _info.num_subcores, _info.num_lanes
_NW = _NC * _NS
_BPW = BATCH // _NW
_CH = 128
_NCH = _BPW // _CH


def _tt_kernel(uid_hbm, iid_hbm, ut_hbm, it_hbm, out_hbm,
               uids_v, iids_v, urows0, urows1, irows0, irows1,
               out_v, usem0, usem1, isem0, isem1):
    wid = lax.axis_index("s") * _NC + lax.axis_index("c")
    base = wid * _BPW
    pltpu.sync_copy(uid_hbm.at[pl.ds(base, _BPW)], uids_v)
    pltpu.sync_copy(iid_hbm.at[pl.ds(base, _BPW)], iids_v)

    ubufs, ibufs = (urows0, urows1), (irows0, irows1)
    usems, isems = (usem0, usem1), (isem0, isem1)
    row_iota = lax.iota(jnp.int32, _L)

    def enqueue(c, s):
        ubuf, ibuf, usem, isem = ubufs[s], ibufs[s], usems[s], isems[s]

        def body(g, carry):
            uvec = uids_v[pl.ds(c * _CH + g * _L, _L)]
            ivec = iids_v[pl.ds(c * _CH + g * _L, _L)]
            for j in range(_L):
                pltpu.make_async_copy(
                    ut_hbm.at[pl.ds(uvec[j], 1), :],
                    ubuf.at[pl.ds(g * _L + j, 1), :], usem).start()
                pltpu.make_async_copy(
                    it_hbm.at[pl.ds(ivec[j], 1), :],
                    ibuf.at[pl.ds(g * _L + j, 1), :], isem).start()
            return carry

        lax.fori_loop(0, _CH // _L, body, 0)

    def wait(s):
        pltpu.make_async_copy(ut_hbm.at[pl.ds(0, _CH), :], ubufs[s],
                              usems[s]).wait()
        pltpu.make_async_copy(it_hbm.at[pl.ds(0, _CH), :], ibufs[s],
                              isems[s]).wait()

    def compute(c, s):
        ubuf, ibuf = ubufs[s], ibufs[s]

        def group_body(g, carry):
            rows = g * _L + row_iota

            def d_body(d, acc):
                col = jnp.full((_L,), d, jnp.int32)
                u = plsc.load_gather(ubuf, [rows, col])
                v = plsc.load_gather(ibuf, [rows, col])
                return acc + u * v

            acc = lax.fori_loop(0, D, d_body, jnp.zeros((_L,), jnp.float32),
                                unroll=8)
            out_v[pl.ds(c * _CH + g * _L, _L)] = acc
            return carry

        lax.fori_loop(0, _CH // _L, group_body, 0)

    enqueue(0, 0)
    for c in range(_NCH):
        s = c % 2
        if c + 1 < _NCH:
            enqueue(c + 1, (c + 1) % 2)
        wait(s)
        compute(c, s)

    pltpu.sync_copy(out_v, out_hbm.at[pl.ds(base, _BPW)])


@jax.jit
def _two_tower(user_ids, item_ids, user_table, item_table):
    mesh = plsc.VectorSubcoreMesh(core_axis_name="c", subcore_axis_name="s")
    f = functools.partial(
        pl.kernel,
        out_type=jax.ShapeDtypeStruct((BATCH,), jnp.float32),
        mesh=mesh,
        scratch_types=[
            pltpu.VMEM((_BPW,), jnp.int32),
            pltpu.VMEM((_BPW,), jnp.int32),
            pltpu.VMEM((_CH, D), jnp.float32),
            pltpu.VMEM((_CH, D), jnp.float32),
            pltpu.VMEM((_CH, D), jnp.float32),
            pltpu.VMEM((_CH, D), jnp.float32),
            pltpu.VMEM((_BPW,), jnp.float32),
            pltpu.SemaphoreType.DMA,
            pltpu.SemaphoreType.DMA,
            pltpu.SemaphoreType.DMA,
            pltpu.SemaphoreType.DMA,
        ],
        compiler_params=pltpu.CompilerParams(needs_layout_passes=False),
    )(_tt_kernel)
    return f(user_ids, item_ids, user_table, item_table)


def kernel(user_ids, item_ids, user_table, item_table):
    return _two_tower(user_ids.astype(jnp.int32), item_ids.astype(jnp.int32),
                      user_table, item_table)
